# baseline (speedup 1.0000x reference)
import functools
import jax
import jax.numpy as jnp
from jax import lax
from jax.experimental import pallas as pl
from jax.experimental.pallas import tpu as pltpu
from jax.experimental.pallas import tpu_sc as plsc

_R = 2048


def _tc_dense_body(logits_ref, t_ref, y_ref, wg_ref, out_ref):
    x = logits_ref[...]
    t = t_ref[0, 0, :]
    yv = y_ref[0, 0, :]
    wg = wg_ref[0, 0, :]
    sel = lax.broadcasted_iota(jnp.int32, x.shape, 0) == t[None, :]
    s = jnp.sum(jnp.exp(x), axis=0)
    g = jnp.sum(jnp.where(sel, x, 0.0), axis=0)
    nll = jnp.log(s * wg) - g
    out_ref[0, 0, :] = jnp.where(jnp.isnan(yv), 0.0, nll)


def _sc_body(nbars, chunk, nc, y_hbm, borders_hbm, t_hbm, wg_hbm,
             y_v, borders_v, t_v, wg_v):
    wid = lax.axis_index("s") * nc + lax.axis_index("c")
    base = wid * chunk
    pltpu.sync_copy(y_hbm.at[pl.ds(base, chunk)], y_v)
    pltpu.sync_copy(borders_hbm, borders_v.at[pl.ds(0, nbars + 1)])

    def body(i, carry):
        yv = y_v[pl.ds(i * 16, 16)]
        lo = jnp.zeros((16,), jnp.int32)
        hi = jnp.full((16,), nbars + 1, jnp.int32)
        for _ in range(7):
            mid = jnp.minimum(lax.shift_right_logical(lo + hi, 1), nbars)
            b = plsc.load_gather(borders_v, [mid])
            p = b < yv
            lo = jnp.where(p, mid + 1, lo)
            hi = jnp.where(p, hi, mid)
        t = jnp.clip(lo - 1, 0, nbars - 1)
        blo = plsc.load_gather(borders_v, [t])
        bhi = plsc.load_gather(borders_v, [t + 1])
        wg_v[pl.ds(i * 16, 16)] = bhi - blo
        t_v[pl.ds(i * 16, 16)] = t
        return carry

    lax.fori_loop(0, chunk // 16, body, 0)
    pltpu.sync_copy(t_v, t_hbm.at[wid])
    pltpu.sync_copy(wg_v, wg_hbm.at[wid])


def kernel(logits, y, borders):
    B, T, K = logits.shape
    N = B * T
    info = plsc.get_sparse_core_info()
    NC, NS = info.num_cores, info.num_subcores
    NW = NC * NS
    chunk = N // NW
    yf = y.reshape(N)
    mesh = plsc.VectorSubcoreMesh(core_axis_name="c", subcore_axis_name="s")
    sc = pl.kernel(
        functools.partial(_sc_body, K, chunk, NC),
        out_type=(jax.ShapeDtypeStruct((NW, chunk), jnp.int32),
                  jax.ShapeDtypeStruct((NW, chunk), jnp.float32)),
        mesh=mesh,
        compiler_params=pltpu.CompilerParams(needs_layout_passes=False,
                                             use_tc_tiling_on_sc=False),
        scratch_types=[
            pltpu.VMEM((chunk,), jnp.float32),
            pltpu.VMEM((128,), jnp.float32),
            pltpu.VMEM((chunk,), jnp.int32),
            pltpu.VMEM((chunk,), jnp.float32),
        ],
    )
    t, wg = sc(yf, borders)

    nb = N // _R
    t3 = t.reshape(nb, 1, _R)
    wg3 = wg.reshape(nb, 1, _R)
    y3 = y.reshape(nb, 1, _R)
    xt = logits.transpose(2, 0, 1).reshape(K, N)
    out = pl.pallas_call(
        _tc_dense_body,
        grid=(nb,),
        in_specs=[pl.BlockSpec((K, _R), lambda i: (0, i)),
                  pl.BlockSpec((1, 1, _R), lambda i: (i, 0, 0)),
                  pl.BlockSpec((1, 1, _R), lambda i: (i, 0, 0)),
                  pl.BlockSpec((1, 1, _R), lambda i: (i, 0, 0))],
        out_specs=pl.BlockSpec((1, 1, _R), lambda i: (i, 0, 0)),
        out_shape=jax.ShapeDtypeStruct((nb, 1, _R), jnp.float32),
    )(xt, t3, y3, wg3)
    return out.reshape(B, T)


# speedup vs baseline: 1.0614x; 1.0614x over previous
import functools
import jax
import jax.numpy as jnp
from jax import lax
from jax.experimental import pallas as pl
from jax.experimental.pallas import tpu as pltpu
from jax.experimental.pallas import tpu_sc as plsc

_R = 4096


def _tc_dense_body(logits_ref, t_ref, y_ref, wg_ref, out_ref):
    x = logits_ref[...]
    t = t_ref[0, 0, :]
    yv = y_ref[0, 0, :]
    wg = wg_ref[0, 0, :]
    sel = lax.broadcasted_iota(jnp.int32, x.shape, 0) == t[None, :]
    s = jnp.sum(jnp.exp(x), axis=0)
    g = jnp.sum(jnp.where(sel, x, 0.0), axis=0)
    nll = jnp.log(s * wg) - g
    out_ref[0, 0, :] = jnp.where(jnp.isnan(yv), 0.0, nll)


def _sc_body(nbars, chunk, nc, y_hbm, borders_hbm, t_hbm, wg_hbm,
             y_v, borders_v, t_v, wg_v):
    wid = lax.axis_index("s") * nc + lax.axis_index("c")
    base = wid * chunk
    pltpu.sync_copy(y_hbm.at[pl.ds(base, chunk)], y_v)
    pltpu.sync_copy(borders_hbm, borders_v.at[pl.ds(0, nbars + 1)])

    def body(i, carry):
        yv = y_v[pl.ds(i * 16, 16)]
        lo = jnp.zeros((16,), jnp.int32)
        hi = jnp.full((16,), nbars + 1, jnp.int32)
        for _ in range(7):
            mid = jnp.minimum(lax.shift_right_logical(lo + hi, 1), nbars)
            b = plsc.load_gather(borders_v, [mid])
            p = b < yv
            lo = jnp.where(p, mid + 1, lo)
            hi = jnp.where(p, hi, mid)
        t = jnp.clip(lo - 1, 0, nbars - 1)
        blo = plsc.load_gather(borders_v, [t])
        bhi = plsc.load_gather(borders_v, [t + 1])
        wg_v[pl.ds(i * 16, 16)] = bhi - blo
        t_v[pl.ds(i * 16, 16)] = t
        return carry

    lax.fori_loop(0, chunk // 16, body, 0)
    pltpu.sync_copy(t_v, t_hbm.at[wid])
    pltpu.sync_copy(wg_v, wg_hbm.at[wid])


def kernel(logits, y, borders):
    B, T, K = logits.shape
    N = B * T
    info = plsc.get_sparse_core_info()
    NC, NS = info.num_cores, info.num_subcores
    NW = NC * NS
    chunk = N // NW
    yf = y.reshape(N)
    mesh = plsc.VectorSubcoreMesh(core_axis_name="c", subcore_axis_name="s")
    sc = pl.kernel(
        functools.partial(_sc_body, K, chunk, NC),
        out_type=(jax.ShapeDtypeStruct((NW, chunk), jnp.int32),
                  jax.ShapeDtypeStruct((NW, chunk), jnp.float32)),
        mesh=mesh,
        compiler_params=pltpu.CompilerParams(needs_layout_passes=False,
                                             use_tc_tiling_on_sc=False),
        scratch_types=[
            pltpu.VMEM((chunk,), jnp.float32),
            pltpu.VMEM((128,), jnp.float32),
            pltpu.VMEM((chunk,), jnp.int32),
            pltpu.VMEM((chunk,), jnp.float32),
        ],
    )
    t, wg = sc(yf, borders)

    nb = N // _R
    t3 = t.reshape(nb, 1, _R)
    wg3 = wg.reshape(nb, 1, _R)
    y3 = y.reshape(nb, 1, _R)
    xt = logits.transpose(2, 0, 1).reshape(K, N)
    out = pl.pallas_call(
        _tc_dense_body,
        grid=(nb,),
        in_specs=[pl.BlockSpec((K, _R), lambda i: (0, i)),
                  pl.BlockSpec((1, 1, _R), lambda i: (i, 0, 0)),
                  pl.BlockSpec((1, 1, _R), lambda i: (i, 0, 0)),
                  pl.BlockSpec((1, 1, _R), lambda i: (i, 0, 0))],
        out_specs=pl.BlockSpec((1, 1, _R), lambda i: (i, 0, 0)),
        out_shape=jax.ShapeDtypeStruct((nb, 1, _R), jnp.float32),
    )(xt, t3, y3, wg3)
    return out.reshape(B, T)


# speedup vs baseline: 1.0976x; 1.0341x over previous
import functools
import jax
import jax.numpy as jnp
from jax import lax
from jax.experimental import pallas as pl
from jax.experimental.pallas import tpu as pltpu
from jax.experimental.pallas import tpu_sc as plsc

_R = 8192


def _tc_dense_body(logits_ref, t_ref, y_ref, wg_ref, out_ref):
    x = logits_ref[...]
    t = t_ref[0, 0, :]
    yv = y_ref[0, 0, :]
    wg = wg_ref[0, 0, :]
    sel = lax.broadcasted_iota(jnp.int32, x.shape, 0) == t[None, :]
    s = jnp.sum(jnp.exp(x), axis=0)
    g = jnp.sum(jnp.where(sel, x, 0.0), axis=0)
    nll = jnp.log(s * wg) - g
    out_ref[0, 0, :] = jnp.where(jnp.isnan(yv), 0.0, nll)


def _sc_body(nbars, chunk, nc, y_hbm, borders_hbm, t_hbm, wg_hbm,
             y_v, borders_v, t_v, wg_v):
    wid = lax.axis_index("s") * nc + lax.axis_index("c")
    base = wid * chunk
    pltpu.sync_copy(y_hbm.at[pl.ds(base, chunk)], y_v)
    pltpu.sync_copy(borders_hbm, borders_v.at[pl.ds(0, nbars + 1)])

    def body(i, carry):
        yv = y_v[pl.ds(i * 16, 16)]
        lo = jnp.zeros((16,), jnp.int32)
        hi = jnp.full((16,), nbars + 1, jnp.int32)
        for _ in range(7):
            mid = jnp.minimum(lax.shift_right_logical(lo + hi, 1), nbars)
            b = plsc.load_gather(borders_v, [mid])
            p = b < yv
            lo = jnp.where(p, mid + 1, lo)
            hi = jnp.where(p, hi, mid)
        t = jnp.clip(lo - 1, 0, nbars - 1)
        blo = plsc.load_gather(borders_v, [t])
        bhi = plsc.load_gather(borders_v, [t + 1])
        wg_v[pl.ds(i * 16, 16)] = bhi - blo
        t_v[pl.ds(i * 16, 16)] = t
        return carry

    lax.fori_loop(0, chunk // 16, body, 0)
    pltpu.sync_copy(t_v, t_hbm.at[wid])
    pltpu.sync_copy(wg_v, wg_hbm.at[wid])


def kernel(logits, y, borders):
    B, T, K = logits.shape
    N = B * T
    info = plsc.get_sparse_core_info()
    NC, NS = info.num_cores, info.num_subcores
    NW = NC * NS
    chunk = N // NW
    yf = y.reshape(N)
    mesh = plsc.VectorSubcoreMesh(core_axis_name="c", subcore_axis_name="s")
    sc = pl.kernel(
        functools.partial(_sc_body, K, chunk, NC),
        out_type=(jax.ShapeDtypeStruct((NW, chunk), jnp.int32),
                  jax.ShapeDtypeStruct((NW, chunk), jnp.float32)),
        mesh=mesh,
        compiler_params=pltpu.CompilerParams(needs_layout_passes=False,
                                             use_tc_tiling_on_sc=False),
        scratch_types=[
            pltpu.VMEM((chunk,), jnp.float32),
            pltpu.VMEM((128,), jnp.float32),
            pltpu.VMEM((chunk,), jnp.int32),
            pltpu.VMEM((chunk,), jnp.float32),
        ],
    )
    t, wg = sc(yf, borders)

    nb = N // _R
    t3 = t.reshape(nb, 1, _R)
    wg3 = wg.reshape(nb, 1, _R)
    y3 = y.reshape(nb, 1, _R)
    xt = logits.transpose(2, 0, 1).reshape(K, N)
    out = pl.pallas_call(
        _tc_dense_body,
        grid=(nb,),
        in_specs=[pl.BlockSpec((K, _R), lambda i: (0, i)),
                  pl.BlockSpec((1, 1, _R), lambda i: (i, 0, 0)),
                  pl.BlockSpec((1, 1, _R), lambda i: (i, 0, 0)),
                  pl.BlockSpec((1, 1, _R), lambda i: (i, 0, 0))],
        out_specs=pl.BlockSpec((1, 1, _R), lambda i: (i, 0, 0)),
        out_shape=jax.ShapeDtypeStruct((nb, 1, _R), jnp.float32),
    )(xt, t3, y3, wg3)
    return out.reshape(B, T)
